# baseline (speedup 1.0000x reference)
import jax
import jax.numpy as jnp
from jax.experimental import pallas as pl

_TIMESTEP = 5
_VTH = 1.0
_TAU = 0.5
_BETA = 0.2

_B = 80
_C, _H, _W = 192, 32, 32
_P = _C * _H * _W
_K = int(_BETA * _P)
_BS = _B // _TIMESTEP
_HW = _H * _W


def _fused_body(x_ref, o_ref):
    xb = x_ref[...].reshape(_TIMESTEP, _HW, _C)
    b = jax.lax.bitcast_convert_type(xb, jnp.int32)
    k1 = jnp.where(b < 0, b ^ jnp.int32(0x7FFFFFFF), b) >> 1
    cnt0 = _P + jnp.sum(k1 >> 31, axis=(1, 2), keepdims=True)
    thr = jnp.where(cnt0 >= _K, jnp.int32(0), jnp.int32(-(1 << 30)))
    for bit in range(29, -1, -1):
        cand = thr + jnp.int32(1 << bit)
        cnt = _P + jnp.sum((k1 - cand) >> 31, axis=(1, 2), keepdims=True)
        thr = jnp.where(cnt >= _K, cand, thr)
    hi1 = (thr << 1) + 1
    hi1_f = jax.lax.bitcast_convert_type(
        jnp.where(hi1 < 0, hi1 ^ jnp.int32(0x7FFFFFFF), hi1), jnp.float32)
    cnt1 = jnp.sum((xb >= hi1_f).astype(jnp.int32), axis=(1, 2),
                   keepdims=True)
    kth = jnp.where(cnt1 >= _K, hi1, thr << 1)
    kth_f = jax.lax.bitcast_convert_type(
        jnp.where(kth < 0, kth ^ jnp.int32(0x7FFFFFFF), kth), jnp.float32)
    u = jnp.zeros((_HW, _C), jnp.float32)
    for t in range(_TIMESTEP):
        mask = (xb[t] >= kth_f[t]).astype(jnp.float32)
        spk_prev = (u > _VTH).astype(jnp.float32)
        u = _TAU * u * (1.0 - spk_prev) + xb[t]
        s = (u > _VTH).astype(jnp.float32)
        o_ref[t, 0] = s * mask


def kernel(x):
    xp = x.transpose(0, 2, 3, 1).reshape(_TIMESTEP, _BS, _HW, _C)
    out = pl.pallas_call(
        _fused_body,
        grid=(_BS,),
        in_specs=[pl.BlockSpec((_TIMESTEP, 1, _HW, _C),
                               lambda j: (0, j, 0, 0))],
        out_specs=pl.BlockSpec((_TIMESTEP, 1, _HW, _C),
                               lambda j: (0, j, 0, 0)),
        out_shape=jax.ShapeDtypeStruct((_TIMESTEP, _BS, _HW, _C),
                                       jnp.float32),
    )(xp)
    return out.reshape(_B, _H, _W, _C).transpose(0, 3, 1, 2)
